# baseline (speedup 1.0000x reference)
import jax
import jax.numpy as jnp
from jax import lax
from jax.experimental import pallas as pl
from jax.experimental.pallas import tpu as pltpu
from jax.experimental.pallas import tpu_sc as plsc

NW = 32
LANES = 16
CHUNK = 4096


def _body(xt_hbm, params_hbm, table_hbm, out_hbm,
          xb0, xb1, yb0, yb1, ob0, ob1, table_v, params_v,
          isx0, isx1, isy0, isy1, osem0, osem1):
    n = xt_hbm.shape[1]
    rows_w = n // NW
    n_chunks = rows_w // CHUNK
    xbs = (xb0, xb1)
    ybs = (yb0, yb1)
    obs = (ob0, ob1)
    isxs = (isx0, isx1)
    isys = (isy0, isy1)
    osems = (osem0, osem1)

    wid = lax.axis_index("s") * 2 + lax.axis_index("c")
    row0 = wid * rows_w

    pltpu.sync_copy(table_hbm, table_v)
    pltpu.sync_copy(params_hbm, params_v)
    axv = params_v[0]
    ayv = params_v[1]
    bxv = params_v[2]
    byv = params_v[3]

    def start_in(j):
        b = j % 2
        col = row0 + j * CHUNK
        dx = pltpu.async_copy(xt_hbm.at[0, pl.ds(col, CHUNK)], xbs[b], isxs[b])
        dy = pltpu.async_copy(xt_hbm.at[1, pl.ds(col, CHUNK)], ybs[b], isys[b])
        return (dx, dy)

    def start_out(j):
        b = j % 2
        return pltpu.async_copy(
            obs[b], out_hbm.at[pl.ds(row0 + j * CHUNK, CHUNK)], osems[b])

    in_d = [None, None]
    out_d = [None, None]
    in_d[0] = start_in(0)
    for j in range(n_chunks):
        b = j % 2
        if j + 1 < n_chunks:
            in_d[1 - b] = start_in(j + 1)
        in_d[b][0].wait()
        in_d[b][1].wait()
        if out_d[b] is not None:
            out_d[b].wait()
        xb = xbs[b]
        yb = ybs[b]
        ob = obs[b]

        @plsc.parallel_loop(0, CHUNK // LANES, 1, unroll=8)
        def _grp(i):
            xv = xb[pl.ds(i * LANES, LANES)]
            yv = yb[pl.ds(i * LANES, LANES)]
            tx = jnp.minimum(jnp.maximum(xv * axv + bxv, 0.0), 31.0)
            ty = jnp.minimum(jnp.maximum(yv * ayv + byv, 0.0), 31.0)
            ix = tx.astype(jnp.int32)
            iy = ty.astype(jnp.int32)
            bins = ix * 32 + iy
            ob[pl.ds(i * LANES, LANES)] = plsc.load_gather(table_v, [bins])

        out_d[b] = start_out(j)
    out_d[0].wait()
    out_d[1].wait()


def kernel(x, mean, std, xedges, yedges, bin_accuracies):
    n = x.shape[0]
    nbx = xedges.shape[0] - 1
    nby = yedges.shape[0] - 1

    inv_x = nbx / (xedges[-1] - xedges[0])
    inv_y = nby / (yedges[-1] - yedges[0])
    ax = inv_x / std[0, 0]
    ay = inv_y / std[0, 1]
    bx = (-mean[0, 0] / std[0, 0] - xedges[0]) * inv_x
    by = (-mean[0, 1] / std[0, 1] - yedges[0]) * inv_y
    ones = jnp.ones((LANES,), jnp.float32)
    params = jnp.stack([ax * ones, ay * ones, bx * ones, by * ones])

    xt = x.T

    mesh = plsc.VectorSubcoreMesh(core_axis_name="c", subcore_axis_name="s")
    f = pl.kernel(
        _body,
        out_type=jax.ShapeDtypeStruct((n,), jnp.float32),
        mesh=mesh,
        compiler_params=pltpu.CompilerParams(needs_layout_passes=False),
        scratch_types=[
            pltpu.VMEM((CHUNK,), jnp.float32),
            pltpu.VMEM((CHUNK,), jnp.float32),
            pltpu.VMEM((CHUNK,), jnp.float32),
            pltpu.VMEM((CHUNK,), jnp.float32),
            pltpu.VMEM((CHUNK,), jnp.float32),
            pltpu.VMEM((CHUNK,), jnp.float32),
            pltpu.VMEM((1024,), jnp.float32),
            pltpu.VMEM((4, LANES), jnp.float32),
            pltpu.SemaphoreType.DMA,
            pltpu.SemaphoreType.DMA,
            pltpu.SemaphoreType.DMA,
            pltpu.SemaphoreType.DMA,
            pltpu.SemaphoreType.DMA,
            pltpu.SemaphoreType.DMA,
        ],
    )
    return f(xt, params, bin_accuracies)


# speedup vs baseline: 1.0502x; 1.0502x over previous
import jax
import jax.numpy as jnp
from jax import lax
from jax.experimental import pallas as pl
from jax.experimental.pallas import tpu as pltpu
from jax.experimental.pallas import tpu_sc as plsc

NW = 32
LANES = 16
CHUNK = 8192


def _body(xt_hbm, params_hbm, table_hbm, out_hbm,
          xb0, xb1, yb0, yb1, ob0, ob1, table_v, params_v,
          isx0, isx1, isy0, isy1, osem0, osem1):
    n = xt_hbm.shape[1]
    rows_w = n // NW
    n_chunks = rows_w // CHUNK
    xbs = (xb0, xb1)
    ybs = (yb0, yb1)
    obs = (ob0, ob1)
    isxs = (isx0, isx1)
    isys = (isy0, isy1)
    osems = (osem0, osem1)

    wid = lax.axis_index("s") * 2 + lax.axis_index("c")
    row0 = wid * rows_w

    pltpu.sync_copy(table_hbm, table_v)
    pltpu.sync_copy(params_hbm, params_v)
    axv = params_v[0]
    ayv = params_v[1]
    bxv = params_v[2]
    byv = params_v[3]

    def start_in(j):
        b = j % 2
        col = row0 + j * CHUNK
        dx = pltpu.async_copy(xt_hbm.at[0, pl.ds(col, CHUNK)], xbs[b], isxs[b])
        dy = pltpu.async_copy(xt_hbm.at[1, pl.ds(col, CHUNK)], ybs[b], isys[b])
        return (dx, dy)

    def start_out(j):
        b = j % 2
        return pltpu.async_copy(
            obs[b], out_hbm.at[pl.ds(row0 + j * CHUNK, CHUNK)], osems[b])

    in_d = [None, None]
    out_d = [None, None]
    in_d[0] = start_in(0)
    for j in range(n_chunks):
        b = j % 2
        if j + 1 < n_chunks:
            in_d[1 - b] = start_in(j + 1)
        in_d[b][0].wait()
        in_d[b][1].wait()
        if out_d[b] is not None:
            out_d[b].wait()
        xb = xbs[b]
        yb = ybs[b]
        ob = obs[b]

        @plsc.parallel_loop(0, CHUNK // LANES, 1, unroll=8)
        def _grp(i):
            xv = xb[pl.ds(i * LANES, LANES)]
            yv = yb[pl.ds(i * LANES, LANES)]
            tx = jnp.minimum(jnp.maximum(xv * axv + bxv, 0.0), 31.0)
            ty = jnp.minimum(jnp.maximum(yv * ayv + byv, 0.0), 31.0)
            ix = tx.astype(jnp.int32)
            iy = ty.astype(jnp.int32)
            bins = ix * 32 + iy
            ob[pl.ds(i * LANES, LANES)] = plsc.load_gather(table_v, [bins])

        out_d[b] = start_out(j)
    out_d[0].wait()
    out_d[1].wait()


def kernel(x, mean, std, xedges, yedges, bin_accuracies):
    n = x.shape[0]
    nbx = xedges.shape[0] - 1
    nby = yedges.shape[0] - 1

    inv_x = nbx / (xedges[-1] - xedges[0])
    inv_y = nby / (yedges[-1] - yedges[0])
    ax = inv_x / std[0, 0]
    ay = inv_y / std[0, 1]
    bx = (-mean[0, 0] / std[0, 0] - xedges[0]) * inv_x
    by = (-mean[0, 1] / std[0, 1] - yedges[0]) * inv_y
    ones = jnp.ones((LANES,), jnp.float32)
    params = jnp.stack([ax * ones, ay * ones, bx * ones, by * ones])

    xt = x.T

    mesh = plsc.VectorSubcoreMesh(core_axis_name="c", subcore_axis_name="s")
    f = pl.kernel(
        _body,
        out_type=jax.ShapeDtypeStruct((n,), jnp.float32),
        mesh=mesh,
        compiler_params=pltpu.CompilerParams(needs_layout_passes=False),
        scratch_types=[
            pltpu.VMEM((CHUNK,), jnp.float32),
            pltpu.VMEM((CHUNK,), jnp.float32),
            pltpu.VMEM((CHUNK,), jnp.float32),
            pltpu.VMEM((CHUNK,), jnp.float32),
            pltpu.VMEM((CHUNK,), jnp.float32),
            pltpu.VMEM((CHUNK,), jnp.float32),
            pltpu.VMEM((1024,), jnp.float32),
            pltpu.VMEM((4, LANES), jnp.float32),
            pltpu.SemaphoreType.DMA,
            pltpu.SemaphoreType.DMA,
            pltpu.SemaphoreType.DMA,
            pltpu.SemaphoreType.DMA,
            pltpu.SemaphoreType.DMA,
            pltpu.SemaphoreType.DMA,
        ],
    )
    return f(xt, params, bin_accuracies)
